# Initial kernel scaffold; baseline (speedup 1.0000x reference)
import jax
import jax.numpy as jnp
from jax.experimental import pallas as pl


def kernel(x, x_0, W, b, edge_index):
    raise NotImplementedError("write your pallas kernel here")


# speedup vs baseline: 16.7845x; 16.7845x over previous
import functools

import jax
import jax.numpy as jnp
from jax import lax
from jax.experimental import pallas as pl
from jax.experimental.pallas import tpu as pltpu
from jax.experimental.pallas import tpu_sc as plsc

_B, _C, _N, _K = 1, 128, 10000, 32
_OUT = 128
_NTILES = 32
_CPT = _C // _NTILES
_CHUNK = 400
_NGROUPS = _CHUNK // 16
_NCHUNKS = _N // _CHUNK
_NEG = -1e30


def _sc_max_relative(xt, e0t, e1t):
    mesh = plsc.VectorSubcoreMesh(core_axis_name="c", subcore_axis_name="s")

    @functools.partial(
        pl.kernel,
        out_type=jax.ShapeDtypeStruct((_C, _N), jnp.float32),
        mesh=mesh,
        scratch_types=[
            pltpu.VMEM((_CPT, _N), jnp.float32),
            pltpu.VMEM((2, _K, _CHUNK), jnp.int32),
            pltpu.VMEM((_CPT, _CHUNK), jnp.float32),
        ],
        compiler_params=pltpu.CompilerParams(
            use_tc_tiling_on_sc=False, needs_layout_passes=False),
    )
    def sc_kernel(xt_hbm, e0_hbm, e1_hbm, out_hbm, xt_v, idx_v, out_v):
        wid = lax.axis_index("s") * 2 + lax.axis_index("c")
        c0 = wid * _CPT
        pltpu.sync_copy(xt_hbm.at[pl.ds(c0, _CPT), :], xt_v)

        def chunk_body(ci, carry):
            col = ci * _CHUNK
            pltpu.sync_copy(e0_hbm.at[:, pl.ds(col, _CHUNK)], idx_v.at[0])
            pltpu.sync_copy(e1_hbm.at[:, pl.ds(col, _CHUNK)], idx_v.at[1])

            def group_body(g, gcarry):
                base = g * 16
                accs = [jnp.full((16,), _NEG, jnp.float32) for _ in range(_CPT)]
                for kk in range(_K):
                    i0 = idx_v[0, kk, pl.ds(base, 16)]
                    i1 = idx_v[1, kk, pl.ds(base, 16)]
                    valid = i0 != i1
                    for c in range(_CPT):
                        csp = jnp.full((16,), c, jnp.int32)
                        xj = plsc.load_gather(xt_v, [csp, i0])
                        xi = plsc.load_gather(xt_v, [csp, i1])
                        d = jnp.where(valid, xj - xi, _NEG)
                        accs[c] = jnp.maximum(accs[c], d)
                for c in range(_CPT):
                    out_v[c, pl.ds(base, 16)] = accs[c]
                return gcarry

            lax.fori_loop(0, _NGROUPS, group_body, 0)
            pltpu.sync_copy(out_v, out_hbm.at[pl.ds(c0, _CPT), pl.ds(col, _CHUNK)])
            return carry

        lax.fori_loop(0, _NCHUNKS, chunk_body, 0)

    return sc_kernel(xt, e0t, e1t)


_BN = 1024


def _tc_conv(xt, m, W, b2):
    def body(w_ref, b_ref, x_ref, m_ref, o_ref):
        acc = jnp.dot(w_ref[:, :_C], x_ref[...],
                      preferred_element_type=jnp.float32)
        acc = acc + jnp.dot(w_ref[:, _C:], m_ref[...],
                            preferred_element_type=jnp.float32)
        o_ref[...] = jnp.maximum(acc + b_ref[...], 0.0)

    grid = (pl.cdiv(_N, _BN),)
    return pl.pallas_call(
        body,
        grid=grid,
        in_specs=[
            pl.BlockSpec((_OUT, 2 * _C), lambda i: (0, 0)),
            pl.BlockSpec((_OUT, 1), lambda i: (0, 0)),
            pl.BlockSpec((_C, _BN), lambda i: (0, i)),
            pl.BlockSpec((_C, _BN), lambda i: (0, i)),
        ],
        out_specs=pl.BlockSpec((_OUT, _BN), lambda i: (0, i)),
        out_shape=jax.ShapeDtypeStruct((_OUT, _N), jnp.float32),
    )(W, b2, xt, m)


def kernel(x, x_0, W, b, edge_index):
    xt = x[0, :, :, 0]
    e = edge_index.astype(jnp.int32)
    e0t = jnp.transpose(e[0, 0])
    e1t = jnp.transpose(e[1, 0])
    m = _sc_max_relative(xt, e0t, e1t)
    out = _tc_conv(xt, m, W, b.reshape(_OUT, 1))
    return out[None, :, :, None]
